# Initial kernel scaffold; baseline (speedup 1.0000x reference)
import jax
import jax.numpy as jnp
from jax.experimental import pallas as pl


def kernel(fp2_features, local_grasp_features, local_color_features, grasp_pose_feature, fp2_xyz, batch_grasp_preds, seed_idxs):
    raise NotImplementedError("write your pallas kernel here")


# speedup vs baseline: 1.2981x; 1.2981x over previous
import dataclasses

import jax
import jax.numpy as jnp
from jax import lax
from jax.experimental import pallas as pl
from jax.experimental.pallas import tpu as pltpu
from jax.experimental.pallas import tpu_sc as plsc

_B, _C, _NS, _K = 8, 256, 16384, 2048
_L = 16
_G = _K // _L
_CPQ = _C // 4
_CH = (13, 14, 15, 4, 5, 6, 7, 8, 9, 10, 11, 12)


def _sc_body(fg, fc, fp, ff, xyz_cm, preds_cm, idx_hbm,
             og, oc, op, of, oxyz_cm, ograsp_cm,
             idx_v, row_v, ob_v):
    cid = lax.axis_index("c")
    sid = lax.axis_index("s")
    wid = sid * 2 + cid
    b = wid // 4
    q = wid % 4

    pltpu.sync_copy(idx_hbm.at[pl.ds(b * _K, _K)], idx_v)

    def gather_row(src_row_ref, dst_row_ref):
        pltpu.sync_copy(src_row_ref, row_v)

        @pl.loop(0, _G)
        def _(g):
            iv = idx_v[pl.ds(g * _L, _L)]
            ob_v[pl.ds(g * _L, _L)] = plsc.load_gather(row_v, [iv])

        pltpu.sync_copy(ob_v, dst_row_ref)

    @pl.when(q == 1)
    def _():
        for d in range(3):
            gather_row(xyz_cm.at[d, b], oxyz_cm.at[d, b])

    @pl.when(q == 2)
    def _():
        for jo, ch in enumerate(_CH):
            gather_row(preds_cm.at[ch, b], ograsp_cm.at[jo, b])

    for feat, out in ((fg, og), (fc, oc), (fp, op), (ff, of)):
        @pl.loop(0, _CPQ)
        def _(r, feat=feat, out=out):
            c = q * _CPQ + r
            gather_row(feat.at[b, c], out.at[b, c])


def kernel(fp2_features, local_grasp_features, local_color_features,
           grasp_pose_feature, fp2_xyz, batch_grasp_preds, seed_idxs):
    idx = seed_idxs.reshape(_B * _K)
    xyz_cm = jnp.transpose(fp2_xyz, (2, 0, 1))
    preds_cm = jnp.transpose(batch_grasp_preds, (2, 0, 1))

    f32 = jnp.float32
    out_type = (
        jax.ShapeDtypeStruct((_B, _C, _K), f32),
        jax.ShapeDtypeStruct((_B, _C, _K), f32),
        jax.ShapeDtypeStruct((_B, _C, _K), f32),
        jax.ShapeDtypeStruct((_B, _C, _K), f32),
        jax.ShapeDtypeStruct((3, _B, _K), f32),
        jax.ShapeDtypeStruct((12, _B, _K), f32),
    )
    scratch = [
        pltpu.VMEM((_K,), jnp.int32),
        pltpu.VMEM((_NS,), f32),
        pltpu.VMEM((_K,), f32),
    ]
    cp = pltpu.CompilerParams()
    if "needs_layout_passes" in pltpu.CompilerParams.__dataclass_fields__:
        cp = dataclasses.replace(cp, needs_layout_passes=False)
    run = pl.kernel(
        _sc_body,
        out_type=out_type,
        mesh=plsc.VectorSubcoreMesh(core_axis_name="c", subcore_axis_name="s"),
        scratch_types=scratch,
        compiler_params=cp,
    )
    og, oc, op, of, oxyz_cm, ograsp_cm = run(
        local_grasp_features, local_color_features, grasp_pose_feature,
        fp2_features, xyz_cm, preds_cm, idx)
    return (og, oc, op, of,
            jnp.transpose(oxyz_cm, (1, 2, 0)),
            jnp.transpose(ograsp_cm, (1, 0, 2)))
